# baseline (speedup 1.0000x reference)
import functools

import jax
import jax.numpy as jnp
from jax.experimental import pallas as pl
from jax.experimental.pallas import tpu as pltpu

B, N, S, D = 8, 4096, 1024, 128
H0, H1 = 256, 128
TN = 4096
NT = N // TN
INV_BN = 1.0 / (1.0 + 1e-05) ** 0.5


def _body(x1_ref, x2t_ref, p1_ref, p2_ref, w0_ref, b0_ref, g0_ref,
          beta0_ref, w1_ref, b1_ref, g1_ref, beta1_ref, out_ref, acc_ref):
    t = pl.program_id(1)

    x1 = x1_ref[0]
    x2t = x2t_ref[0]

    cross = jnp.dot(x2t, -2.0 * x1, preferred_element_type=jnp.float32)
    x1sq = jnp.sum(x1 * x1, axis=0, keepdims=True)
    x2sq = jnp.sum(x2t * x2t, axis=1, keepdims=True)
    dist = (cross + x1sq) + x2sq

    d = dist
    hots, mins = [], []
    for k in range(3):
        m = jnp.min(d, axis=0, keepdims=True)
        hot = d == m
        hots.append(hot)
        mins.append(m)
        if k < 2:
            d = jnp.where(hot, jnp.inf, d)

    packed = (jnp.where(hots[0], 1.0, 0.0) + jnp.where(hots[1], 256.0, 0.0)
              + jnp.where(hots[2], 65536.0, 0.0))
    cnt = jnp.sum(packed, axis=0, keepdims=True)
    c2 = jnp.floor(cnt * (1.0 / 65536.0))
    rem = cnt - c2 * 65536.0
    c1 = jnp.floor(rem * (1.0 / 256.0))
    c0 = rem - c1 * 256.0

    take0 = jnp.minimum(c0, 3.0)
    take1 = jnp.minimum(c1, 3.0 - take0)
    take2 = jnp.maximum(jnp.minimum(c2, 3.0 - take0 - take1), 0.0)
    r0 = 1.0 / (mins[0] + 1e-08)
    r1 = 1.0 / (mins[1] + 1e-08)
    r2 = 1.0 / (mins[2] + 1e-08)
    total = r0 * take0 + r1 * take1 + r2 * take2
    wmat = (jnp.where(hots[0], r0 * (take0 / c0), 0.0)
            + jnp.where(hots[1], r1 * (take1 / c1), 0.0)
            + jnp.where(hots[2], r2 * (take2 / c2), 0.0))

    interp = jnp.dot(p2_ref[0], wmat, preferred_element_type=jnp.float32)
    interp = interp * (1.0 / total)

    w0 = w0_ref[...]
    contrib = jnp.concatenate(
        [jnp.dot(p1_ref[0], w0, preferred_element_type=jnp.float32),
         jnp.dot(interp, w0, preferred_element_type=jnp.float32)], axis=0)

    @pl.when(t == 0)
    def _():
        acc_ref[...] = jnp.zeros_like(acc_ref)

    acc_ref[...] += contrib

    @pl.when(t == NT - 1)
    def _():
        h = acc_ref[...] + b0_ref[...]
        h = jnp.maximum(h * INV_BN * g0_ref[...] + beta0_ref[...], 0.0)
        h = jnp.dot(h, w1_ref[...], preferred_element_type=jnp.float32)
        h = h + b1_ref[...]
        h = jnp.maximum(h * INV_BN * g1_ref[...] + beta1_ref[...], 0.0)
        out_ref[0] = h


@jax.jit
def kernel(xyz1, xyz2, points1, points2, W0, b0, g0, beta0, W1, b1, g1, beta1):
    xyz2t = jnp.transpose(xyz2, (0, 2, 1))

    grid = (B, NT)
    out = pl.pallas_call(
        _body,
        grid=grid,
        in_specs=[
            pl.BlockSpec((1, 3, TN), lambda b, t: (b, 0, t)),
            pl.BlockSpec((1, S, 3), lambda b, t: (b, 0, 0)),
            pl.BlockSpec((1, D, TN), lambda b, t: (b, 0, t)),
            pl.BlockSpec((1, D, S), lambda b, t: (b, 0, 0)),
            pl.BlockSpec((TN, H0), lambda b, t: (t, 0)),
            pl.BlockSpec((1, H0), lambda b, t: (0, 0)),
            pl.BlockSpec((1, H0), lambda b, t: (0, 0)),
            pl.BlockSpec((1, H0), lambda b, t: (0, 0)),
            pl.BlockSpec((H0, H1), lambda b, t: (0, 0)),
            pl.BlockSpec((1, H1), lambda b, t: (0, 0)),
            pl.BlockSpec((1, H1), lambda b, t: (0, 0)),
            pl.BlockSpec((1, H1), lambda b, t: (0, 0)),
        ],
        out_specs=pl.BlockSpec((1, 2 * D, H1), lambda b, t: (b, 0, 0)),
        out_shape=jax.ShapeDtypeStruct((B, 2 * D, H1), jnp.float32),
        scratch_shapes=[pltpu.VMEM((2 * D, H0), jnp.float32)],
    )(xyz1, xyz2t, points1, points2, W0,
      b0.reshape(1, H0), g0.reshape(1, H0), beta0.reshape(1, H0),
      W1, b1.reshape(1, H1), g1.reshape(1, H1), beta1.reshape(1, H1))
    return out


# speedup vs baseline: 1.0278x; 1.0278x over previous
import jax
import jax.numpy as jnp
from jax.experimental import pallas as pl
from jax.experimental.pallas import tpu as pltpu

B, N, S, D = 8, 4096, 1024, 128
H0, H1 = 256, 128
TN = 4096
NT = N // TN
INV_BN = 1.0 / (1.0 + 1e-05) ** 0.5


def _body(x1_ref, x2t_ref, p1_ref, p2_ref, w0_ref, b0_ref, g0_ref,
          beta0_ref, w1_ref, b1_ref, g1_ref, beta1_ref, out_ref, acc_ref):
    t = pl.program_id(1)

    x1 = x1_ref[0]
    x2t = x2t_ref[0]

    cross = jnp.dot(x2t, -2.0 * x1, preferred_element_type=jnp.float32)
    x1sq = jnp.sum(x1 * x1, axis=0, keepdims=True)
    x2sq = jnp.sum(x2t * x2t, axis=1, keepdims=True)
    dist = (cross + x1sq) + x2sq

    d = dist
    rank = jnp.zeros((1, TN), jnp.float32)
    total = jnp.zeros((1, TN), jnp.float32)
    wmat = jnp.zeros((S, TN), jnp.float32)
    for k in range(3):
        m = jnp.min(d, axis=0, keepdims=True)
        hot = d == m
        hotf = hot.astype(jnp.float32)
        c = jnp.sum(hotf, axis=0, keepdims=True)
        take = jnp.minimum(c, 3.0 - rank)
        r = 1.0 / (m + 1e-08)
        wmat = wmat + hotf * (r * (take / c))
        total = total + r * take
        rank = rank + take
        if k < 2:
            d = jnp.where(hot, jnp.inf, d)

    interp = jnp.dot(p2_ref[0], wmat, preferred_element_type=jnp.float32)
    interp = interp * (1.0 / total)

    w0 = w0_ref[...]
    contrib = jnp.concatenate(
        [jnp.dot(p1_ref[0], w0, preferred_element_type=jnp.float32),
         jnp.dot(interp, w0, preferred_element_type=jnp.float32)], axis=0)

    @pl.when(t == 0)
    def _():
        acc_ref[...] = jnp.zeros_like(acc_ref)

    acc_ref[...] += contrib

    @pl.when(t == NT - 1)
    def _():
        h = acc_ref[...] + b0_ref[...]
        h = jnp.maximum(h * INV_BN * g0_ref[...] + beta0_ref[...], 0.0)
        h = jnp.dot(h, w1_ref[...], preferred_element_type=jnp.float32)
        h = h + b1_ref[...]
        h = jnp.maximum(h * INV_BN * g1_ref[...] + beta1_ref[...], 0.0)
        out_ref[0] = h


@jax.jit
def kernel(xyz1, xyz2, points1, points2, W0, b0, g0, beta0, W1, b1, g1, beta1):
    xyz2t = jnp.transpose(xyz2, (0, 2, 1))

    grid = (B, NT)
    out = pl.pallas_call(
        _body,
        grid=grid,
        in_specs=[
            pl.BlockSpec((1, 3, TN), lambda b, t: (b, 0, t)),
            pl.BlockSpec((1, S, 3), lambda b, t: (b, 0, 0)),
            pl.BlockSpec((1, D, TN), lambda b, t: (b, 0, t)),
            pl.BlockSpec((1, D, S), lambda b, t: (b, 0, 0)),
            pl.BlockSpec((TN, H0), lambda b, t: (t, 0)),
            pl.BlockSpec((1, H0), lambda b, t: (0, 0)),
            pl.BlockSpec((1, H0), lambda b, t: (0, 0)),
            pl.BlockSpec((1, H0), lambda b, t: (0, 0)),
            pl.BlockSpec((H0, H1), lambda b, t: (0, 0)),
            pl.BlockSpec((1, H1), lambda b, t: (0, 0)),
            pl.BlockSpec((1, H1), lambda b, t: (0, 0)),
            pl.BlockSpec((1, H1), lambda b, t: (0, 0)),
        ],
        out_specs=pl.BlockSpec((1, 2 * D, H1), lambda b, t: (b, 0, 0)),
        out_shape=jax.ShapeDtypeStruct((B, 2 * D, H1), jnp.float32),
        scratch_shapes=[pltpu.VMEM((2 * D, H0), jnp.float32)],
    )(xyz1, xyz2t, points1, points2, W0,
      b0.reshape(1, H0), g0.reshape(1, H0), beta0.reshape(1, H0),
      W1, b1.reshape(1, H1), g1.reshape(1, H1), beta1.reshape(1, H1))
    return out
